# baseline (speedup 1.0000x reference)
import jax
import jax.numpy as jnp
from jax import lax
from jax.experimental import pallas as pl
from jax.experimental.pallas import tpu as pltpu
from jax.experimental.pallas import tpu_sc as plsc

N = 10000
E = 320000
H = 128
OUT = 40
G = 64
NLAYERS = 3
BN_EPS = 1e-5

NC = 2
NS = 16
NW = NC * NS
CHUNK = 128
TILE_EDGES = E // NW
TILE_PAD = 10240
NCHUNK = TILE_PAD // CHUNK
PHC = NCHUNK // 2
NPAD = 10240
STRIPE = NPAD // NS
NB = 5120
GRID = NPAD // NB


def _sc_agg_body(tab, src_h, dst_h, zer, out,
                 src_v, dst_v, rows0, rows1, acc, sem0, sem1, sem2, sem3):
    c = lax.axis_index("c")
    s = lax.axis_index("s")
    wid = c * NS + s
    pltpu.sync_copy(zer.at[pl.ds(s * STRIPE, STRIPE)],
                    acc.at[pl.ds(s * STRIPE, STRIPE)])
    pltpu.sync_copy(src_h.at[pl.ds(wid * TILE_PAD, PHC * CHUNK)], src_v)
    pltpu.sync_copy(dst_h.at[wid], dst_v)
    plsc.subcore_barrier()

    def sidx(gl):
        return src_v.at[pl.ds(gl * CHUNK, CHUNK)]

    def gwait0():
        pltpu.make_async_copy(tab.at[sidx(0)], rows0, sem0).wait()

    def gwait1():
        pltpu.make_async_copy(tab.at[sidx(0)], rows1, sem1).wait()

    def swait0():
        pltpu.make_async_copy(rows0, acc.at[dst_v.at[0]], sem2).wait()

    def swait1():
        pltpu.make_async_copy(rows1, acc.at[dst_v.at[0]], sem3).wait()

    for p in range(2):
        if p == 1:
            pltpu.sync_copy(
                src_h.at[pl.ds(wid * TILE_PAD + PHC * CHUNK, PHC * CHUNK)],
                src_v)
        base = p * PHC
        pltpu.async_copy(tab.at[sidx(0)], rows0, sem0)
        pltpu.async_copy(tab.at[sidx(1)], rows1, sem1)

        def step(k, carry, base=base):
            g = 2 * k
            gwait0()
            pltpu.async_copy(rows0, acc.at[dst_v.at[base + g]], sem2, add=True)
            gwait1()
            pltpu.async_copy(rows1, acc.at[dst_v.at[base + g + 1]], sem3,
                             add=True)
            swait0()
            pltpu.async_copy(tab.at[sidx(g + 2)], rows0, sem0)
            swait1()
            pltpu.async_copy(tab.at[sidx(g + 3)], rows1, sem1)
            return carry

        lax.fori_loop(0, (PHC - 2) // 2, step, 0)
        gwait0()
        pltpu.async_copy(rows0, acc.at[dst_v.at[base + PHC - 2]], sem2,
                         add=True)
        gwait1()
        pltpu.async_copy(rows1, acc.at[dst_v.at[base + PHC - 1]], sem3,
                         add=True)
        swait0()
        swait1()

    plsc.subcore_barrier()
    pltpu.sync_copy(acc.at[pl.ds(s * STRIPE, STRIPE)],
                    out.at[c, pl.ds(s * STRIPE, STRIPE)])


_sc_agg = pl.kernel(
    _sc_agg_body,
    out_type=jax.ShapeDtypeStruct((NC, NPAD, H), jnp.float32),
    mesh=plsc.VectorSubcoreMesh(core_axis_name="c", subcore_axis_name="s"),
    scratch_types=[
        pltpu.VMEM((PHC * CHUNK,), jnp.int32),
        pltpu.VMEM((NCHUNK, CHUNK), jnp.int32),
        pltpu.VMEM((CHUNK, H), jnp.float32),
        pltpu.VMEM((CHUNK, H), jnp.float32),
        pltpu.VMEM_SHARED((NPAD, H), jnp.float32),
        pltpu.SemaphoreType.DMA,
        pltpu.SemaphoreType.DMA,
        pltpu.SemaphoreType.DMA,
        pltpu.SemaphoreType.DMA,
    ],
)


DW = 128
CHUNKD = 128
NCHUNKD = 79


def _sc_deg_body(dst_h, onesr_h, zer_h, out, dst_v, rows, acc, sem0, sem1):
    c = lax.axis_index("c")
    s = lax.axis_index("s")
    wid = c * NS + s
    pltpu.sync_copy(zer_h.at[pl.ds(s * STRIPE, STRIPE)],
                    acc.at[pl.ds(s * STRIPE, STRIPE)])
    pltpu.sync_copy(dst_h.at[wid], dst_v)
    pltpu.sync_copy(onesr_h, rows)
    plsc.subcore_barrier()

    def step(k, carry):
        g = 2 * k
        pltpu.async_copy(rows, acc.at[dst_v.at[g]], sem0, add=True)
        pltpu.async_copy(rows, acc.at[dst_v.at[g + 1]], sem1, add=True)
        pltpu.make_async_copy(rows, acc.at[dst_v.at[0]], sem0).wait()
        pltpu.make_async_copy(rows, acc.at[dst_v.at[0]], sem1).wait()
        return carry

    lax.fori_loop(0, NCHUNKD // 2, step, 0)
    pltpu.sync_copy(rows, acc.at[dst_v.at[NCHUNKD - 1]], add=True)
    plsc.subcore_barrier()
    pltpu.sync_copy(acc.at[pl.ds(s * STRIPE, STRIPE)],
                    out.at[c, pl.ds(s * STRIPE, STRIPE)])


_sc_deg = pl.kernel(
    _sc_deg_body,
    out_type=jax.ShapeDtypeStruct((NC, NPAD, DW), jnp.float32),
    mesh=plsc.VectorSubcoreMesh(core_axis_name="c", subcore_axis_name="s"),
    scratch_types=[
        pltpu.VMEM((NCHUNKD, CHUNKD), jnp.int32),
        pltpu.VMEM((CHUNKD, DW), jnp.float32),
        pltpu.VMEM_SHARED((NPAD, DW), jnp.float32),
        pltpu.SemaphoreType.DMA,
        pltpu.SemaphoreType.DMA,
    ],
)


def _lin_body(h_ref, w_ref, b_ref, o_ref):
    o_ref[...] = (
        jnp.dot(h_ref[...], w_ref[...], preferred_element_type=jnp.float32)
        + b_ref[...]
    )


_lin = pl.pallas_call(
    _lin_body,
    grid=(GRID,),
    in_specs=[
        pl.BlockSpec((NB, H), lambda i: (i, 0)),
        pl.BlockSpec((H, H), lambda i: (0, 0)),
        pl.BlockSpec((1, H), lambda i: (0, 0)),
    ],
    out_specs=pl.BlockSpec((NB, H), lambda i: (i, 0)),
    out_shape=jax.ShapeDtypeStruct((NPAD, H), jnp.float32),
)


def _sage_update(p_ref, pc_ref, h_ref, wl_ref, bl_ref, gm_ref, bt_ref):
    ssum = p_ref[0] + p_ref[1]
    cnt = (pc_ref[0] + pc_ref[1])[:, 0:1]
    mean = ssum / jnp.maximum(cnt, 1.0)
    out = mean + jnp.dot(h_ref[...], wl_ref[...],
                         preferred_element_type=jnp.float32) + bl_ref[...]
    nrm = jnp.sqrt(jnp.sum(out * out, axis=1, keepdims=True))
    out = out / jnp.maximum(nrm, 1e-12)
    out = out * (gm_ref[...] / jnp.sqrt(1.0 + BN_EPS)) + bt_ref[...]
    return jnp.maximum(out, 0.0)


def _upd_lin_body(p_ref, pc_ref, h_ref, wl_ref, bl_ref, gm_ref, bt_ref,
                  wr_ref, br_ref, h_out, y_out):
    h = _sage_update(p_ref, pc_ref, h_ref, wl_ref, bl_ref, gm_ref, bt_ref)
    h_out[...] = h
    y_out[...] = jnp.dot(h, wr_ref[...],
                         preferred_element_type=jnp.float32) + br_ref[...]


_vec = pl.BlockSpec((1, H), lambda i: (0, 0))
_mat = pl.BlockSpec((H, H), lambda i: (0, 0))
_row = pl.BlockSpec((NB, H), lambda i: (i, 0))

_upd_lin = pl.pallas_call(
    _upd_lin_body,
    grid=(GRID,),
    in_specs=[
        pl.BlockSpec((NC, NB, H), lambda i: (0, i, 0)),
        pl.BlockSpec((NC, NB, DW), lambda i: (0, i, 0)),
        _row, _mat, _vec, _vec, _vec, _mat, _vec,
    ],
    out_specs=(_row, _row),
    out_shape=(jax.ShapeDtypeStruct((NPAD, H), jnp.float32),
               jax.ShapeDtypeStruct((NPAD, H), jnp.float32)),
)


def _upd_head_body(p_ref, pc_ref, h_ref, wl_ref, bl_ref, gm_ref, bt_ref,
                   w1_ref, b1_ref, w2_ref, b2_ref, b_ref, o_ref):
    i = pl.program_id(0)
    h = _sage_update(p_ref, pc_ref, h_ref, wl_ref, bl_ref, gm_ref, bt_ref)
    z = jnp.dot(h, w1_ref[...], preferred_element_type=jnp.float32) + b1_ref[...]
    z = jnp.dot(z, w2_ref[...], preferred_element_type=jnp.float32) + b2_ref[...]

    @pl.when(i == 0)
    def _():
        o_ref[...] = jnp.zeros_like(o_ref)

    onehot = (b_ref[...] == lax.broadcasted_iota(jnp.int32, (1, G), 1)
              ).astype(jnp.float32)
    o_ref[...] += lax.dot_general(onehot, z, (((0,), (0,)), ((), ())),
                                  preferred_element_type=jnp.float32)

    @pl.when(i == GRID - 1)
    def _():
        x = o_ref[...]
        col = lax.broadcasted_iota(jnp.int32, (G, H), 1)
        valid = col < OUT
        m = jnp.max(jnp.where(valid, x, jnp.float32(-1e30)),
                    axis=1, keepdims=True)
        e = jnp.where(valid, jnp.exp(x - m), 0.0)
        o_ref[...] = x - m - jnp.log(jnp.sum(e, axis=1, keepdims=True))


_upd_head = pl.pallas_call(
    _upd_head_body,
    grid=(GRID,),
    in_specs=[
        pl.BlockSpec((NC, NB, H), lambda i: (0, i, 0)),
        pl.BlockSpec((NC, NB, DW), lambda i: (0, i, 0)),
        _row, _mat, _vec, _vec, _vec, _mat, _vec, _mat, _vec,
        pl.BlockSpec((NB, 1), lambda i: (i, 0)),
    ],
    out_specs=pl.BlockSpec((G, H), lambda i: (0, 0)),
    out_shape=jax.ShapeDtypeStruct((G, H), jnp.float32),
)


def kernel(x, edge_index, batch, lin_l_w, lin_l_b, lin_r_w, lin_r_b,
           bn_gamma, bn_beta, W1, b1, W2, b2):
    padw = TILE_PAD - TILE_EDGES
    src2 = jnp.concatenate(
        [edge_index[0].reshape(NW, TILE_EDGES),
         jnp.zeros((NW, padw), jnp.int32)], axis=1).reshape(NW * TILE_PAD)
    dst2 = jnp.concatenate(
        [edge_index[1].reshape(NW, TILE_EDGES),
         jnp.broadcast_to(N + jnp.arange(padw, dtype=jnp.int32) % (NPAD - N),
                          (NW, padw))],
        axis=1).reshape(NW, NCHUNK, CHUNK)
    zeros = jnp.zeros((NPAD, H), jnp.float32)
    x = jnp.concatenate([x, jnp.zeros((NPAD - N, H), jnp.float32)])
    batch2 = jnp.concatenate(
        [batch, jnp.full((NPAD - N,), G, jnp.int32)]).reshape(NPAD, 1)
    W2p = jnp.zeros((H, H), jnp.float32).at[:, :OUT].set(W2)
    b2p = jnp.zeros((1, H), jnp.float32).at[0, :OUT].set(b2)

    padd = NCHUNKD * CHUNKD - TILE_EDGES
    dstd = jnp.concatenate(
        [edge_index[1].reshape(NW, TILE_EDGES),
         jnp.broadcast_to(N + jnp.arange(padd, dtype=jnp.int32) % (NPAD - N),
                          (NW, padd))],
        axis=1).reshape(NW, NCHUNKD, CHUNKD)
    pc = _sc_deg(dstd, jnp.ones((CHUNKD, DW), jnp.float32),
                 jnp.zeros((NPAD, DW), jnp.float32))
    h = x
    y = _lin(x, lin_r_w[0], lin_r_b[0].reshape(1, H))
    for i in range(NLAYERS - 1):
        p = _sc_agg(y, src2, dst2, zeros)
        h, y = _upd_lin(p, pc, h, lin_l_w[i], lin_l_b[i].reshape(1, H),
                        bn_gamma[i].reshape(1, H), bn_beta[i].reshape(1, H),
                        lin_r_w[i + 1], lin_r_b[i + 1].reshape(1, H))
    p = _sc_agg(y, src2, dst2, zeros)
    i = NLAYERS - 1
    pooled = _upd_head(p, pc, h, lin_l_w[i], lin_l_b[i].reshape(1, H),
                       bn_gamma[i].reshape(1, H), bn_beta[i].reshape(1, H),
                       W1, b1.reshape(1, H), W2p, b2p, batch2)
    return pooled[:, :OUT]


# speedup vs baseline: 2.3148x; 2.3148x over previous
import jax
import jax.numpy as jnp
from jax import lax
from jax.experimental import pallas as pl
from jax.experimental.pallas import tpu as pltpu
from jax.experimental.pallas import tpu_sc as plsc

N = 10000
E = 320000
H = 128
OUT = 40
G = 64
NLAYERS = 3
BN_EPS = 1e-5

NC = 2
NS = 16
NW = NC * NS
CHUNK = 80
TILE_EDGES = E // NW
NCHUNK = TILE_EDGES // CHUNK
NPAD = 10240
STRIPE = NPAD // NS
NB = 5120
GRID = NPAD // NB


def _sc_agg_body(tab, src_h, dst_h, zer, out,
                 src_v, dst_v, rows0, rows1, acc, sem0, sem1, sem2, sem3):
    c = lax.axis_index("c")
    s = lax.axis_index("s")
    wid = c * NS + s
    pltpu.sync_copy(zer.at[pl.ds(s * STRIPE, STRIPE)],
                    acc.at[pl.ds(s * STRIPE, STRIPE)])
    pltpu.sync_copy(src_h.at[pl.ds(wid * TILE_EDGES, TILE_EDGES)], src_v)
    pltpu.sync_copy(dst_h.at[wid], dst_v)
    plsc.subcore_barrier()

    def sidx(g):
        return src_v.at[pl.ds(g * CHUNK, CHUNK)]

    def gwait0():
        pltpu.make_async_copy(tab.at[sidx(0)], rows0, sem0).wait()

    def gwait1():
        pltpu.make_async_copy(tab.at[sidx(0)], rows1, sem1).wait()

    def swait0():
        pltpu.make_async_copy(rows0, acc.at[dst_v.at[0]], sem2).wait()

    def swait1():
        pltpu.make_async_copy(rows1, acc.at[dst_v.at[0]], sem3).wait()

    pltpu.async_copy(tab.at[sidx(0)], rows0, sem0)
    pltpu.async_copy(tab.at[sidx(1)], rows1, sem1)

    def step(k, carry):
        g = 2 * k
        gwait0()
        pltpu.async_copy(rows0, acc.at[dst_v.at[g]], sem2, add=True)
        gwait1()
        pltpu.async_copy(rows1, acc.at[dst_v.at[g + 1]], sem3, add=True)
        swait0()
        pltpu.async_copy(tab.at[sidx(g + 2)], rows0, sem0)
        swait1()
        pltpu.async_copy(tab.at[sidx(g + 3)], rows1, sem1)
        return carry

    lax.fori_loop(0, (NCHUNK - 3) // 2, step, 0)
    gwait0()
    pltpu.async_copy(rows0, acc.at[dst_v.at[NCHUNK - 3]], sem2, add=True)
    gwait1()
    pltpu.async_copy(rows1, acc.at[dst_v.at[NCHUNK - 2]], sem3, add=True)
    swait0()
    pltpu.async_copy(tab.at[sidx(NCHUNK - 1)], rows0, sem0)
    swait1()
    gwait0()
    pltpu.sync_copy(rows0, acc.at[dst_v.at[NCHUNK - 1]], add=True)
    plsc.subcore_barrier()
    pltpu.sync_copy(acc.at[pl.ds(s * STRIPE, STRIPE)],
                    out.at[c, pl.ds(s * STRIPE, STRIPE)])


_sc_agg = pl.kernel(
    _sc_agg_body,
    out_type=jax.ShapeDtypeStruct((NC, NPAD, H), jnp.float32),
    mesh=plsc.VectorSubcoreMesh(core_axis_name="c", subcore_axis_name="s"),
    scratch_types=[
        pltpu.VMEM((TILE_EDGES,), jnp.int32),
        pltpu.VMEM((NCHUNK, CHUNK), jnp.int32),
        pltpu.VMEM((CHUNK, H), jnp.float32),
        pltpu.VMEM((CHUNK, H), jnp.float32),
        pltpu.VMEM_SHARED((NPAD, H), jnp.float32),
        pltpu.SemaphoreType.DMA,
        pltpu.SemaphoreType.DMA,
        pltpu.SemaphoreType.DMA,
        pltpu.SemaphoreType.DMA,
    ],
)


DW = 128
CHUNKD = 128
NCHUNKD = 79


def _sc_deg_body(dst_h, onesr_h, zer_h, out, dst_v, rows, acc, sem0, sem1):
    c = lax.axis_index("c")
    s = lax.axis_index("s")
    wid = c * NS + s
    pltpu.sync_copy(zer_h.at[pl.ds(s * STRIPE, STRIPE)],
                    acc.at[pl.ds(s * STRIPE, STRIPE)])
    pltpu.sync_copy(dst_h.at[wid], dst_v)
    pltpu.sync_copy(onesr_h, rows)
    plsc.subcore_barrier()

    def step(k, carry):
        g = 2 * k
        pltpu.async_copy(rows, acc.at[dst_v.at[g]], sem0, add=True)
        pltpu.async_copy(rows, acc.at[dst_v.at[g + 1]], sem1, add=True)
        pltpu.make_async_copy(rows, acc.at[dst_v.at[0]], sem0).wait()
        pltpu.make_async_copy(rows, acc.at[dst_v.at[0]], sem1).wait()
        return carry

    lax.fori_loop(0, NCHUNKD // 2, step, 0)
    pltpu.sync_copy(rows, acc.at[dst_v.at[NCHUNKD - 1]], add=True)
    plsc.subcore_barrier()
    pltpu.sync_copy(acc.at[pl.ds(s * STRIPE, STRIPE)],
                    out.at[c, pl.ds(s * STRIPE, STRIPE)])


_sc_deg = pl.kernel(
    _sc_deg_body,
    out_type=jax.ShapeDtypeStruct((NC, NPAD, DW), jnp.float32),
    mesh=plsc.VectorSubcoreMesh(core_axis_name="c", subcore_axis_name="s"),
    scratch_types=[
        pltpu.VMEM((NCHUNKD, CHUNKD), jnp.int32),
        pltpu.VMEM((CHUNKD, DW), jnp.float32),
        pltpu.VMEM_SHARED((NPAD, DW), jnp.float32),
        pltpu.SemaphoreType.DMA,
        pltpu.SemaphoreType.DMA,
    ],
)


def _lin_body(h_ref, w_ref, b_ref, o_ref):
    o_ref[...] = (
        jnp.dot(h_ref[...], w_ref[...], preferred_element_type=jnp.float32)
        + b_ref[...]
    )


_lin = pl.pallas_call(
    _lin_body,
    grid=(GRID,),
    in_specs=[
        pl.BlockSpec((NB, H), lambda i: (i, 0)),
        pl.BlockSpec((H, H), lambda i: (0, 0)),
        pl.BlockSpec((1, H), lambda i: (0, 0)),
    ],
    out_specs=pl.BlockSpec((NB, H), lambda i: (i, 0)),
    out_shape=jax.ShapeDtypeStruct((NPAD, H), jnp.float32),
)


def _sage_update(p_ref, pc_ref, h_ref, wl_ref, bl_ref, gm_ref, bt_ref):
    ssum = p_ref[0] + p_ref[1]
    cnt = (pc_ref[0] + pc_ref[1])[:, 0:1]
    mean = ssum / jnp.maximum(cnt, 1.0)
    out = mean + jnp.dot(h_ref[...], wl_ref[...],
                         preferred_element_type=jnp.float32) + bl_ref[...]
    nrm = jnp.sqrt(jnp.sum(out * out, axis=1, keepdims=True))
    out = out / jnp.maximum(nrm, 1e-12)
    out = out * (gm_ref[...] / jnp.sqrt(1.0 + BN_EPS)) + bt_ref[...]
    return jnp.maximum(out, 0.0)


def _upd_lin_body(p_ref, pc_ref, h_ref, wl_ref, bl_ref, gm_ref, bt_ref,
                  wr_ref, br_ref, h_out, y_out):
    h = _sage_update(p_ref, pc_ref, h_ref, wl_ref, bl_ref, gm_ref, bt_ref)
    h_out[...] = h
    y_out[...] = jnp.dot(h, wr_ref[...],
                         preferred_element_type=jnp.float32) + br_ref[...]


_vec = pl.BlockSpec((1, H), lambda i: (0, 0))
_mat = pl.BlockSpec((H, H), lambda i: (0, 0))
_row = pl.BlockSpec((NB, H), lambda i: (i, 0))

_upd_lin = pl.pallas_call(
    _upd_lin_body,
    grid=(GRID,),
    in_specs=[
        pl.BlockSpec((NC, NB, H), lambda i: (0, i, 0)),
        pl.BlockSpec((NC, NB, DW), lambda i: (0, i, 0)),
        _row, _mat, _vec, _vec, _vec, _mat, _vec,
    ],
    out_specs=(_row, _row),
    out_shape=(jax.ShapeDtypeStruct((NPAD, H), jnp.float32),
               jax.ShapeDtypeStruct((NPAD, H), jnp.float32)),
)


def _upd_head_body(p_ref, pc_ref, h_ref, wl_ref, bl_ref, gm_ref, bt_ref,
                   w1_ref, b1_ref, w2_ref, b2_ref, b_ref, o_ref):
    i = pl.program_id(0)
    h = _sage_update(p_ref, pc_ref, h_ref, wl_ref, bl_ref, gm_ref, bt_ref)
    z = jnp.dot(h, w1_ref[...], preferred_element_type=jnp.float32) + b1_ref[...]
    z = jnp.dot(z, w2_ref[...], preferred_element_type=jnp.float32) + b2_ref[...]

    @pl.when(i == 0)
    def _():
        o_ref[...] = jnp.zeros_like(o_ref)

    onehot = (b_ref[...] == lax.broadcasted_iota(jnp.int32, (1, G), 1)
              ).astype(jnp.float32)
    o_ref[...] += lax.dot_general(onehot, z, (((0,), (0,)), ((), ())),
                                  preferred_element_type=jnp.float32)

    @pl.when(i == GRID - 1)
    def _():
        x = o_ref[...]
        col = lax.broadcasted_iota(jnp.int32, (G, H), 1)
        valid = col < OUT
        m = jnp.max(jnp.where(valid, x, jnp.float32(-1e30)),
                    axis=1, keepdims=True)
        e = jnp.where(valid, jnp.exp(x - m), 0.0)
        o_ref[...] = x - m - jnp.log(jnp.sum(e, axis=1, keepdims=True))


_upd_head = pl.pallas_call(
    _upd_head_body,
    grid=(GRID,),
    in_specs=[
        pl.BlockSpec((NC, NB, H), lambda i: (0, i, 0)),
        pl.BlockSpec((NC, NB, DW), lambda i: (0, i, 0)),
        _row, _mat, _vec, _vec, _vec, _mat, _vec, _mat, _vec,
        pl.BlockSpec((NB, 1), lambda i: (i, 0)),
    ],
    out_specs=pl.BlockSpec((G, H), lambda i: (0, 0)),
    out_shape=jax.ShapeDtypeStruct((G, H), jnp.float32),
)


def kernel(x, edge_index, batch, lin_l_w, lin_l_b, lin_r_w, lin_r_b,
           bn_gamma, bn_beta, W1, b1, W2, b2):
    src2 = edge_index[0]
    dst2 = edge_index[1].reshape(NW, NCHUNK, CHUNK)
    zeros = jnp.zeros((NPAD, H), jnp.float32)
    x = jnp.concatenate([x, jnp.zeros((NPAD - N, H), jnp.float32)])
    batch2 = jnp.concatenate(
        [batch, jnp.full((NPAD - N,), G, jnp.int32)]).reshape(NPAD, 1)
    W2p = jnp.zeros((H, H), jnp.float32).at[:, :OUT].set(W2)
    b2p = jnp.zeros((1, H), jnp.float32).at[0, :OUT].set(b2)

    padd = NCHUNKD * CHUNKD - TILE_EDGES
    dstd = jnp.concatenate(
        [edge_index[1].reshape(NW, TILE_EDGES),
         jnp.broadcast_to(N + jnp.arange(padd, dtype=jnp.int32) % (NPAD - N),
                          (NW, padd))],
        axis=1).reshape(NW, NCHUNKD, CHUNKD)
    pc = _sc_deg(dstd, jnp.ones((CHUNKD, DW), jnp.float32),
                 jnp.zeros((NPAD, DW), jnp.float32))
    h = x
    y = _lin(x, lin_r_w[0], lin_r_b[0].reshape(1, H))
    for i in range(NLAYERS - 1):
        p = _sc_agg(y, src2, dst2, zeros)
        h, y = _upd_lin(p, pc, h, lin_l_w[i], lin_l_b[i].reshape(1, H),
                        bn_gamma[i].reshape(1, H), bn_beta[i].reshape(1, H),
                        lin_r_w[i + 1], lin_r_b[i + 1].reshape(1, H))
    p = _sc_agg(y, src2, dst2, zeros)
    i = NLAYERS - 1
    pooled = _upd_head(p, pc, h, lin_l_w[i], lin_l_b[i].reshape(1, H),
                       bn_gamma[i].reshape(1, H), bn_beta[i].reshape(1, H),
                       W1, b1.reshape(1, H), W2p, b2p, batch2)
    return pooled[:, :OUT]
